# Initial kernel scaffold; baseline (speedup 1.0000x reference)
import jax
import jax.numpy as jnp
from jax.experimental import pallas as pl


def kernel(h_neigh, h_self, edge_index, W_neigh, W_self):
    raise NotImplementedError("write your pallas kernel here")


# speedup vs baseline: 4.4667x; 4.4667x over previous
import functools
import math

import jax
import jax.numpy as jnp
from jax import lax
from jax.experimental import pallas as pl
from jax.experimental.pallas import tpu as pltpu
from jax.experimental.pallas import tpu_sc as plsc

NC = 2
NS = 16
NW = NC * NS
L = 16
C = 128


def _sc_aggregate(h_pad, src, dst, n_pad, d, chunks_per_worker):
    rows_per_tile = n_pad // NS
    zchunks = rows_per_tile // C
    mesh = plsc.VectorSubcoreMesh(core_axis_name="c", subcore_axis_name="s")

    @functools.partial(
        pl.kernel,
        out_type=(
            jax.ShapeDtypeStruct((NC, n_pad, d), jnp.float32),
            jax.ShapeDtypeStruct((NW, n_pad), jnp.float32),
        ),
        mesh=mesh,
        compiler_params=pltpu.CompilerParams(needs_layout_passes=False),
        scratch_types=[
            pltpu.VMEM((C,), jnp.int32),
            pltpu.VMEM((C,), jnp.int32),
            pltpu.VMEM((C, d), jnp.float32),
            pltpu.VMEM((n_pad,), jnp.float32),
            pltpu.VMEM_SHARED((n_pad, d), jnp.float32),
            pltpu.SemaphoreType.DMA,
        ],
    )
    def agg(h_ref, src_ref, dst_ref, feat_ref, deg_ref, sidx, didx, rows,
            degs, acc, sem):
        cid = lax.axis_index("c")
        sid = lax.axis_index("s")
        wid = cid * NS + sid

        zvec = jnp.zeros((L,), jnp.float32)

        def zrow(i, carry):
            for j in range(d // L):
                rows[i, pl.ds(j * L, L)] = zvec
            return carry

        lax.fori_loop(0, C, zrow, 0)

        def zdeg(i, carry):
            degs[pl.ds(i * L, L)] = zvec
            return carry

        lax.fori_loop(0, n_pad // L, zdeg, 0)

        r0 = sid * rows_per_tile

        def zacc(k, carry):
            pltpu.sync_copy(rows, acc.at[pl.ds(r0 + k * C, C)])
            return carry

        lax.fori_loop(0, zchunks, zacc, 0)
        plsc.subcore_barrier()

        e0 = wid * chunks_per_worker * C
        ones16 = jnp.full((L,), 1.0, jnp.float32)

        def body(j, carry):
            base = e0 + j * C
            pltpu.sync_copy(src_ref.at[pl.ds(base, C)], sidx)
            pltpu.sync_copy(dst_ref.at[pl.ds(base, C)], didx)
            pltpu.async_copy(h_ref.at[sidx], rows, sem).wait()
            pltpu.sync_copy(rows, acc.at[didx], add=True)
            for k in range(C // L):
                dvec = didx[pl.ds(k * L, L)]
                plsc.addupdate_scatter(degs, [dvec], ones16)
            return carry

        lax.fori_loop(0, chunks_per_worker, body, 0)
        plsc.subcore_barrier()

        pltpu.sync_copy(acc.at[pl.ds(r0, rows_per_tile)],
                        feat_ref.at[cid, pl.ds(r0, rows_per_tile)])
        pltpu.sync_copy(degs, deg_ref.at[wid])

    return agg(h_pad, src, dst)


def _tc_finish(partials, deg_all, h_self, W_self, W_neigh, bm):
    n, d = h_self.shape
    out = W_self.shape[1]

    def body(p_ref, dg_ref, hs_ref, ws_ref, wn_ref, o_ref):
        deg = jnp.sum(dg_ref[...], axis=1, keepdims=True)
        neigh = (p_ref[0] + p_ref[1]) / jnp.maximum(deg, 1.0)
        z = (jnp.dot(hs_ref[...], ws_ref[...], preferred_element_type=jnp.float32)
             + jnp.dot(neigh, wn_ref[...], preferred_element_type=jnp.float32))
        z = jnp.maximum(z, 0.0)
        nrm = jnp.sqrt(jnp.sum(z * z, axis=1, keepdims=True))
        nrm = jnp.where(nrm == 0.0, 1.0, nrm)
        o_ref[...] = z / nrm

    return pl.pallas_call(
        body,
        grid=(n // bm,),
        in_specs=[
            pl.BlockSpec((NC, bm, d), lambda i: (0, i, 0)),
            pl.BlockSpec((bm, NW), lambda i: (i, 0)),
            pl.BlockSpec((bm, d), lambda i: (i, 0)),
            pl.BlockSpec((d, out), lambda i: (0, 0)),
            pl.BlockSpec((d, out), lambda i: (0, 0)),
        ],
        out_specs=pl.BlockSpec((bm, out), lambda i: (i, 0)),
        out_shape=jax.ShapeDtypeStruct((n, out), jnp.float32),
    )(partials, deg_all, h_self, W_self, W_neigh)


def kernel(h_neigh, h_self, edge_index, W_neigh, W_self):
    n, d = h_neigh.shape
    e = edge_index.shape[1]

    n_pad = ((n + 1 + NS * C - 1) // (NS * C)) * (NS * C)
    cpw = math.ceil(e / (NW * C))
    e_pad = NW * C * cpw

    src = edge_index[0].astype(jnp.int32)
    dst = edge_index[1].astype(jnp.int32)
    pad = e_pad - e
    if pad:
        fill = jnp.full((pad,), n, jnp.int32)
        src = jnp.concatenate([src, fill])
        dst = jnp.concatenate([dst, fill])

    h_padded = jnp.zeros((n_pad, d), jnp.float32).at[:n].set(h_neigh)

    partials, deg_all = _sc_aggregate(h_padded, src, dst, n_pad, d, cpw)

    bm = next(b for b in (400, 200, 100, 50, 25, 10, 5, 1)
              if n % b == 0 and (b % 8 == 0 or b == n))
    return _tc_finish(partials, deg_all.T, h_self, W_self, W_neigh, bm)


# speedup vs baseline: 4.8359x; 1.0827x over previous
import functools
import math

import jax
import jax.numpy as jnp
from jax import lax
from jax.experimental import pallas as pl
from jax.experimental.pallas import tpu as pltpu
from jax.experimental.pallas import tpu_sc as plsc

NC = 2
NS = 16
NW = NC * NS
L = 16
C = 128


NBUF = 2


def _sc_aggregate(h_pad, sd_idx, n_pad, d, chunks_per_worker):
    rows_per_tile = n_pad // NS
    zchunks = rows_per_tile // C
    cpw = chunks_per_worker
    mesh = plsc.VectorSubcoreMesh(core_axis_name="c", subcore_axis_name="s")

    @functools.partial(
        pl.kernel,
        out_type=(
            jax.ShapeDtypeStruct((NC, n_pad, d), jnp.float32),
            jax.ShapeDtypeStruct((NW, n_pad), jnp.float32),
        ),
        mesh=mesh,
        compiler_params=pltpu.CompilerParams(needs_layout_passes=False),
        scratch_types=[
            [pltpu.VMEM((2, C), jnp.int32) for _ in range(NBUF)],
            [pltpu.VMEM((C, d), jnp.float32) for _ in range(NBUF)],
            pltpu.VMEM((n_pad,), jnp.float32),
            pltpu.VMEM_SHARED((n_pad, d), jnp.float32),
            [pltpu.SemaphoreType.DMA for _ in range(NBUF)],
        ],
    )
    def agg(h_ref, sd_ref, feat_ref, deg_ref, sd, rows, degs, acc, gsem):
        cid = lax.axis_index("c")
        sid = lax.axis_index("s")
        wid = cid * NS + sid

        zvec = jnp.zeros((L,), jnp.float32)

        def zrow(i, carry):
            for j in range(d // L):
                rows[0][i, pl.ds(j * L, L)] = zvec
            return carry

        lax.fori_loop(0, C, zrow, 0)

        def zdeg(i, carry):
            degs[pl.ds(i * L, L)] = zvec
            return carry

        lax.fori_loop(0, n_pad // L, zdeg, 0)

        r0 = sid * rows_per_tile

        def zacc(k, carry):
            pltpu.sync_copy(rows[0], acc.at[pl.ds(r0 + k * C, C)])
            return carry

        lax.fori_loop(0, zchunks, zacc, 0)
        plsc.subcore_barrier()

        ones16 = jnp.full((L,), 1.0, jnp.float32)

        for b in range(NBUF):
            pltpu.sync_copy(sd_ref.at[wid, b], sd[b])
            pltpu.async_copy(h_ref.at[sd[b].at[0]], rows[b], gsem[b])

        def body(p, carry):
            for b in range(NBUF):
                j = p * NBUF + b
                pltpu.make_async_copy(h_ref.at[sd[b].at[0]], rows[b],
                                      gsem[b]).wait()
                pltpu.sync_copy(rows[b], acc.at[sd[b].at[1]], add=True)
                for k in range(C // L):
                    dvec = sd[b][1, pl.ds(k * L, L)]
                    plsc.addupdate_scatter(degs, [dvec], ones16)

                @pl.when(j + NBUF < cpw)
                def _():
                    pltpu.sync_copy(sd_ref.at[wid, j + NBUF], sd[b])
                    pltpu.async_copy(h_ref.at[sd[b].at[0]], rows[b], gsem[b])

            return carry

        lax.fori_loop(0, cpw // NBUF, body, 0)
        plsc.subcore_barrier()

        pltpu.sync_copy(acc.at[pl.ds(r0, rows_per_tile)],
                        feat_ref.at[cid, pl.ds(r0, rows_per_tile)])
        pltpu.sync_copy(degs, deg_ref.at[wid])

    return agg(h_pad, sd_idx)


def _tc_finish(partials, deg_all, h_self, W_self, W_neigh, bm):
    n, d = h_self.shape
    out = W_self.shape[1]

    def body(p_ref, dg_ref, hs_ref, ws_ref, wn_ref, o_ref):
        deg = jnp.sum(dg_ref[...], axis=1, keepdims=True)
        neigh = (p_ref[0] + p_ref[1]) / jnp.maximum(deg, 1.0)
        z = (jnp.dot(hs_ref[...], ws_ref[...], preferred_element_type=jnp.float32)
             + jnp.dot(neigh, wn_ref[...], preferred_element_type=jnp.float32))
        z = jnp.maximum(z, 0.0)
        nrm = jnp.sqrt(jnp.sum(z * z, axis=1, keepdims=True))
        nrm = jnp.where(nrm == 0.0, 1.0, nrm)
        o_ref[...] = z / nrm

    return pl.pallas_call(
        body,
        grid=(n // bm,),
        in_specs=[
            pl.BlockSpec((NC, bm, d), lambda i: (0, i, 0)),
            pl.BlockSpec((bm, NW), lambda i: (i, 0)),
            pl.BlockSpec((bm, d), lambda i: (i, 0)),
            pl.BlockSpec((d, out), lambda i: (0, 0)),
            pl.BlockSpec((d, out), lambda i: (0, 0)),
        ],
        out_specs=pl.BlockSpec((bm, out), lambda i: (i, 0)),
        out_shape=jax.ShapeDtypeStruct((n, out), jnp.float32),
    )(partials, deg_all, h_self, W_self, W_neigh)


def kernel(h_neigh, h_self, edge_index, W_neigh, W_self):
    n, d = h_neigh.shape
    e = edge_index.shape[1]

    n_pad = ((n + 1 + NS * C - 1) // (NS * C)) * (NS * C)
    cpw = ((math.ceil(e / (NW * C)) + NBUF - 1) // NBUF) * NBUF
    e_pad = NW * C * cpw

    src = edge_index[0].astype(jnp.int32)
    dst = edge_index[1].astype(jnp.int32)
    pad = e_pad - e
    if pad:
        fill = jnp.full((pad,), n, jnp.int32)
        src = jnp.concatenate([src, fill])
        dst = jnp.concatenate([dst, fill])
    sd = jnp.stack([src.reshape(NW, cpw, C), dst.reshape(NW, cpw, C)], axis=2)

    h_padded = jnp.zeros((n_pad, d), jnp.float32).at[:n].set(h_neigh)

    partials, deg_all = _sc_aggregate(h_padded, sd, n_pad, d, cpw)

    bm = next(b for b in (400, 200, 100, 50, 25, 10, 5, 1)
              if n % b == 0 and (b % 8 == 0 or b == n))
    return _tc_finish(partials, deg_all.T, h_self, W_self, W_neigh, bm)


# speedup vs baseline: 12.1550x; 2.5135x over previous
import functools
import math

import jax
import jax.numpy as jnp
from jax import lax
from jax.experimental import pallas as pl
from jax.experimental.pallas import tpu as pltpu
from jax.experimental.pallas import tpu_sc as plsc

NC = 2
NS = 16
NW = NC * NS
L = 16
C = 128


NBUF = 2


def _sc_aggregate(h_pad, sd_idx, n_pad, d, chunks_per_worker):
    rows_per_tile = n_pad // NS
    zchunks = rows_per_tile // C
    cpw = chunks_per_worker
    mesh = plsc.VectorSubcoreMesh(core_axis_name="c", subcore_axis_name="s")

    @functools.partial(
        pl.kernel,
        out_type=(
            jax.ShapeDtypeStruct((NC, n_pad, d), jnp.float32),
            jax.ShapeDtypeStruct((NW, n_pad), jnp.float32),
        ),
        mesh=mesh,
        compiler_params=pltpu.CompilerParams(needs_layout_passes=False),
        scratch_types=[
            [pltpu.VMEM((2, C), jnp.int32) for _ in range(NBUF)],
            [pltpu.VMEM((C, d), jnp.float32) for _ in range(NBUF)],
            pltpu.VMEM((n_pad,), jnp.float32),
            pltpu.VMEM_SHARED((n_pad, d), jnp.float32),
            [pltpu.SemaphoreType.DMA for _ in range(NBUF)],
        ],
    )
    def agg(h_ref, sd_ref, feat_ref, deg_ref, sd, rows, degs, acc, gsem):
        cid = lax.axis_index("c")
        sid = lax.axis_index("s")
        wid = cid * NS + sid

        zvec = jnp.zeros((L,), jnp.float32)

        def zrow(i, carry):
            for j in range(d // L):
                rows[0][i, pl.ds(j * L, L)] = zvec
            return carry

        lax.fori_loop(0, C, zrow, 0)

        def zdeg(i, carry):
            degs[pl.ds(i * L, L)] = zvec
            return carry

        lax.fori_loop(0, n_pad // L, zdeg, 0)

        r0 = sid * rows_per_tile

        def zacc(k, carry):
            pltpu.sync_copy(rows[0], acc.at[pl.ds(r0 + k * C, C)])
            return carry

        lax.fori_loop(0, zchunks, zacc, 0)
        plsc.subcore_barrier()

        ones16 = jnp.full((L,), 1.0, jnp.float32)

        for b in range(NBUF):
            pltpu.sync_copy(sd_ref.at[wid, b], sd[b])
            pltpu.async_copy(h_ref.at[sd[b].at[0]], rows[b], gsem[b])

        def body(p, carry):
            for b in range(NBUF):
                j = p * NBUF + b
                pltpu.make_async_copy(h_ref.at[sd[b].at[0]], rows[b],
                                      gsem[b]).wait()
                pltpu.sync_copy(rows[b], acc.at[sd[b].at[1]], add=True)
                for k in range(C // L):
                    dvec = sd[b][1, pl.ds(k * L, L)]
                    plsc.addupdate_scatter(degs, [dvec], ones16)

                @pl.when(j + NBUF < cpw)
                def _():
                    pltpu.sync_copy(sd_ref.at[wid, j + NBUF], sd[b])
                    pltpu.async_copy(h_ref.at[sd[b].at[0]], rows[b], gsem[b])

            return carry

        lax.fori_loop(0, cpw // NBUF, body, 0)
        plsc.subcore_barrier()

        pltpu.sync_copy(acc.at[pl.ds(r0, rows_per_tile)],
                        feat_ref.at[cid, pl.ds(r0, rows_per_tile)])
        pltpu.sync_copy(degs, deg_ref.at[wid])

    return agg(h_pad, sd_idx)


def _tc_finish(partials, deg_all, h_self, W_self, W_neigh, bm):
    n, d = h_self.shape
    out = W_self.shape[1]

    def body(p_ref, dg_ref, hs_ref, ws_ref, wn_ref, o_ref):
        deg = jnp.sum(dg_ref[...], axis=1, keepdims=True)
        neigh = (p_ref[0] + p_ref[1]) / jnp.maximum(deg, 1.0)
        z = (jnp.dot(hs_ref[...], ws_ref[...], preferred_element_type=jnp.float32)
             + jnp.dot(neigh, wn_ref[...], preferred_element_type=jnp.float32))
        z = jnp.maximum(z, 0.0)
        nrm = jnp.sqrt(jnp.sum(z * z, axis=1, keepdims=True))
        nrm = jnp.where(nrm == 0.0, 1.0, nrm)
        o_ref[...] = z / nrm

    return pl.pallas_call(
        body,
        grid=(n // bm,),
        in_specs=[
            pl.BlockSpec((NC, bm, d), lambda i: (0, i, 0)),
            pl.BlockSpec((bm, NW), lambda i: (i, 0)),
            pl.BlockSpec((bm, d), lambda i: (i, 0)),
            pl.BlockSpec((d, out), lambda i: (0, 0)),
            pl.BlockSpec((d, out), lambda i: (0, 0)),
        ],
        out_specs=pl.BlockSpec((bm, out), lambda i: (i, 0)),
        out_shape=jax.ShapeDtypeStruct((n, out), jnp.float32),
    )(partials, deg_all, h_self, W_self, W_neigh)


def kernel(h_neigh, h_self, edge_index, W_neigh, W_self):
    n, d = h_neigh.shape
    e = edge_index.shape[1]

    n_pad = ((n + 1 + NS * C - 1) // (NS * C)) * (NS * C)
    cpw = ((math.ceil(e / (NW * C)) + NBUF - 1) // NBUF) * NBUF
    e_pad = NW * C * cpw

    src = edge_index[0].astype(jnp.int32)
    dst = edge_index[1].astype(jnp.int32)
    pad = e_pad - e
    if pad:
        ar = jnp.arange(pad, dtype=jnp.int32)
        src = jnp.concatenate([src, ar % n])
        dst = jnp.concatenate([dst, n + ar % (n_pad - n)])
    sd = jnp.stack([src.reshape(NW, cpw, C), dst.reshape(NW, cpw, C)], axis=2)

    partials, deg_all = _sc_aggregate(h_neigh, sd, n_pad, d, cpw)

    bm = next(b for b in (400, 200, 100, 50, 25, 10, 5, 1)
              if n % b == 0 and (b % 8 == 0 or b == n))
    return _tc_finish(partials, deg_all.T, h_self, W_self, W_neigh, bm)
